# Initial kernel scaffold; baseline (speedup 1.0000x reference)
import jax
import jax.numpy as jnp
from jax.experimental import pallas as pl


def kernel(x, edge_index, W, att_src, att_dst, bias, ln_gamma, ln_beta):
    raise NotImplementedError("write your pallas kernel here")


# speedup vs baseline: 27.1347x; 27.1347x over previous
import functools

import jax
import jax.numpy as jnp
from jax import lax
from jax.experimental import pallas as pl
from jax.experimental.pallas import tpu as pltpu
from jax.experimental.pallas import tpu_sc as plsc

F32 = jnp.float32
I32 = jnp.int32

G = 4
N = 10000
NP = 10240
D = 128
H = 2
RB = 1280
NRB = NP // RB

NSC = 2
NS = 16
C = 128
EL = 160000 + N
CPT = -(-EL // (NS * C))
EPT = CPT * C
EP = EPT * NS
RPT = NP // NS
RZ = RPT // 5


def _tc1_body(x_ref, w_ref, asrc_ref, adst_ref, xpT_ref, aT_ref, cT_ref):
    r = pl.program_id(1)
    xb = x_ref[0]
    xp = jnp.dot(xb, w_ref[...], preferred_element_type=F32)
    mx = []
    for h in range(H):
        xph = xp[:, h * D:(h + 1) * D]
        xpT_ref[h, 0] = xph
        a_s = (xph * asrc_ref[h][None, :]).sum(axis=1)
        a_d = (xph * adst_ref[h][None, :]).sum(axis=1)
        aT_ref[0, h, 0] = a_s.reshape(RB // D, D)
        aT_ref[1, h, 0] = a_d.reshape(RB // D, D)
        mx.append((jnp.max(a_s), jnp.max(a_d)))

    built = jnp.concatenate(
        [jnp.full((1, D), mx[0][0], F32), jnp.full((1, D), mx[1][0], F32),
         jnp.full((1, D), mx[0][1], F32), jnp.full((1, D), mx[1][1], F32),
         jnp.full((4, D), -1e30, F32)], axis=0)

    @pl.when(r == 0)
    def _():
        cT_ref[0] = jnp.full((8, D), -1e30, F32)
    cT_ref[0] = jnp.maximum(cT_ref[0], built)


_tc1 = pl.pallas_call(
    _tc1_body,
    grid=(G, NRB),
    in_specs=[
        pl.BlockSpec((1, RB, D), lambda g, r: (g, r, 0)),
        pl.BlockSpec((D, H * D), lambda g, r: (0, 0)),
        pl.BlockSpec((H, D), lambda g, r: (0, 0)),
        pl.BlockSpec((H, D), lambda g, r: (0, 0)),
    ],
    out_specs=[
        pl.BlockSpec((H, 1, RB, D), lambda g, r: (0, g, r, 0)),
        pl.BlockSpec((2, H, 1, RB // D, D), lambda g, r: (0, 0, g * NRB + r, 0, 0)),
        pl.BlockSpec((1, 8, D), lambda g, r: (g, 0, 0)),
    ],
    out_shape=[
        jax.ShapeDtypeStruct((H, G, NP, D), F32),
        jax.ShapeDtypeStruct((2, H, G * NRB, RB // D, D), F32),
        jax.ShapeDtypeStruct((G, 8, D), F32),
    ],
)


def _sca_body(aS, aD, cT, srcp, dstp, validp,
              wT,
              src_v, dst_v, valid_v, wv_v, a_s_v, a_d_v, c_v):
    h = lax.axis_index("c")
    t = lax.axis_index("s")

    pltpu.sync_copy(srcp.at[pl.ds(t * EPT, EPT)], src_v)
    pltpu.sync_copy(dstp.at[pl.ds(t * EPT, EPT)], dst_v)
    pltpu.sync_copy(validp.at[pl.ds(t * EPT, EPT)], valid_v)

    for g in range(G):
        pltpu.sync_copy(aS.at[h, g], a_s_v)
        pltpu.sync_copy(aD.at[h, g], a_d_v)
        pltpu.sync_copy(cT.at[g], c_v)
        cs = c_v[h, pl.ds(0, 16)]
        cd = c_v[2 + h, pl.ds(0, 16)]
        csum = cs + cd
        c16 = jnp.where(csum > 0, csum, 0.2 * csum)

        def _grp(i, _):
            sl = pl.ds(i * 16, 16)
            av = plsc.load_gather(a_s_v, [src_v[sl]])
            bv = plsc.load_gather(a_d_v, [dst_v[sl]])
            al = av + bv
            al = jnp.where(al > 0, al, 0.2 * al)
            wv_v[sl] = jnp.exp(al - c16) * valid_v[sl]
            return 0
        lax.fori_loop(0, EPT // 16, _grp, 0)

        pltpu.sync_copy(wv_v, wT.at[h, g].at[pl.ds(t * EPT, EPT)])


_sca = functools.partial(
    pl.kernel,
    out_type=jax.ShapeDtypeStruct((H, G, EP), F32),
    mesh=plsc.VectorSubcoreMesh(core_axis_name="c", subcore_axis_name="s",
                                num_cores=NSC, num_subcores=NS),
    compiler_params=pltpu.CompilerParams(needs_layout_passes=False),
    scratch_types=[
        pltpu.VMEM((EPT,), I32),
        pltpu.VMEM((EPT,), I32),
        pltpu.VMEM((EPT,), F32),
        pltpu.VMEM((EPT,), F32),
        pltpu.VMEM((NP,), F32),
        pltpu.VMEM((NP,), F32),
        pltpu.VMEM((8, D), F32),
    ],
)(_sca_body)


def _scb_body(xpT, srcp2, dstp2, wT,
              numT,
              sidx_v, didx_v, w_v, rows_v, den_v, inv_v,
              num_sh, den_stage, sem):
    h = lax.axis_index("c")
    t = lax.axis_index("s")
    zero16 = jnp.zeros((16,), F32)

    for g in range(G):
        def _z0(i, _):
            rows_v[i // 8, pl.ds((i % 8) * 16, 16)] = zero16
            return 0
        lax.fori_loop(0, C * 8, _z0, 0)

        def _z1(i, _):
            den_v[i // 40, pl.ds((i % 40) * 16, 16)] = zero16
            return 0
        lax.fori_loop(0, NP // 16, _z1, 0)

        for j in range(5):
            sl = pl.ds(t * RPT + j * RZ, RZ)
            pltpu.sync_copy(rows_v, num_sh.at[sl])
        plsc.subcore_barrier()

        def _chunk(k, _):
            row = t * CPT + k
            pltpu.sync_copy(srcp2.at[pl.ds(row, 1)], sidx_v)
            pltpu.sync_copy(dstp2.at[pl.ds(row, 1)], didx_v)
            pltpu.sync_copy(wT.at[h, g].at[pl.ds(row * C, C)], w_v)

            gcp = pltpu.async_copy(xpT.at[h, g].at[sidx_v.at[0]], rows_v, sem)

            def _dc(j, _):
                sl = pl.ds(j * 16, 16)
                di = didx_v[0, sl]
                plsc.addupdate_scatter(den_v, [di // RPT, di % RPT], w_v[sl])
                return 0
            lax.fori_loop(0, C // 16, _dc, 0)

            gcp.wait()

            def _s(e, _):
                ws = plsc.load_gather(w_v, [jnp.zeros((16,), I32) + e])
                for u in range(D // 16):
                    sl = pl.ds(u * 16, 16)
                    rows_v[e, sl] = rows_v[e, sl] * ws
                return 0
            lax.fori_loop(0, C, _s, 0)

            pltpu.sync_copy(rows_v, num_sh.at[didx_v.at[0]], add=True)
            return 0
        lax.fori_loop(0, CPT, _chunk, 0)

        pltpu.sync_copy(den_v, den_stage.at[t])
        plsc.subcore_barrier()
        for j in range(NS):
            pltpu.sync_copy(den_stage.at[j, t], den_v.at[j])

        def _dsum(i, _):
            sl = pl.ds(i * 16, 16)
            acc = den_v[0, sl]
            for j in range(1, NS):
                acc = acc + den_v[j, sl]
            inv_v[sl] = 1.0 / (acc + 1e-16)
            return 0
        lax.fori_loop(0, RPT // 16, _dsum, 0)

        for j in range(5):
            sl = pl.ds(t * RPT + j * RZ, RZ)
            pltpu.sync_copy(num_sh.at[sl], rows_v)

            def _n(e, _):
                ws = plsc.load_gather(inv_v, [jnp.zeros((16,), I32) + (j * RZ + e)])
                for u in range(D // 16):
                    sl2 = pl.ds(u * 16, 16)
                    rows_v[e, sl2] = rows_v[e, sl2] * ws
                return 0
            lax.fori_loop(0, RZ, _n, 0)
            pltpu.sync_copy(rows_v, numT.at[h, g].at[sl])


_scb = functools.partial(
    pl.kernel,
    out_type=jax.ShapeDtypeStruct((H, G, NP, D), F32),
    mesh=plsc.VectorSubcoreMesh(core_axis_name="c", subcore_axis_name="s",
                                num_cores=NSC, num_subcores=NS),
    compiler_params=pltpu.CompilerParams(needs_layout_passes=False),
    scratch_types=[
        pltpu.VMEM((1, C), I32),
        pltpu.VMEM((1, C), I32),
        pltpu.VMEM((C,), F32),
        pltpu.VMEM((C, D), F32),
        pltpu.VMEM((NS, RPT), F32),
        pltpu.VMEM((RPT,), F32),
        pltpu.VMEM_SHARED((NP, D), F32),
        pltpu.VMEM_SHARED((NS, NS, RPT), F32),
        pltpu.SemaphoreType.DMA,
    ],
)(_scb_body)


def _tc2_body(x_ref, num_ref, bias_ref, g_ref, b_ref, out_ref):
    o = 0.5 * (num_ref[0, 0] + num_ref[1, 0]) + bias_ref[0][None, :]
    y = x_ref[0] + o
    mu = jnp.mean(y, axis=1, keepdims=True)
    var = jnp.mean((y - mu) ** 2, axis=1, keepdims=True)
    out_ref[0] = (y - mu) * lax.rsqrt(var + 1e-5) * g_ref[0][None, :] + b_ref[0][None, :]


_tc2 = pl.pallas_call(
    _tc2_body,
    grid=(G, NRB),
    in_specs=[
        pl.BlockSpec((1, RB, D), lambda g, r: (g, r, 0)),
        pl.BlockSpec((H, 1, RB, D), lambda g, r: (0, g, r, 0)),
        pl.BlockSpec((1, D), lambda g, r: (0, 0)),
        pl.BlockSpec((1, D), lambda g, r: (0, 0)),
        pl.BlockSpec((1, D), lambda g, r: (0, 0)),
    ],
    out_specs=pl.BlockSpec((1, RB, D), lambda g, r: (g, r, 0)),
    out_shape=jax.ShapeDtypeStruct((G, NP, D), F32),
)


def kernel(x, edge_index, W, att_src, att_dst, bias, ln_gamma, ln_beta):
    b, t, n, d = x.shape
    x4 = x.reshape(G, N, D)
    xpad = jnp.pad(x4, ((0, 0), (0, NP - N), (0, 0)))

    xpT, aT, cT = _tc1(xpad, W, att_src, att_dst)
    aS = aT[0].reshape(H, G, NP)
    aD = aT[1].reshape(H, G, NP)

    ar = jnp.arange(N, dtype=I32)
    ei = edge_index.astype(I32)
    src = jnp.concatenate([ei[0], ar])
    dst = jnp.concatenate([ei[1], ar])
    srcp = jnp.pad(src, (0, EP - EL))
    dstp = jnp.pad(dst, (0, EP - EL))
    validp = jnp.pad(jnp.ones((EL,), F32), (0, EP - EL))

    wT = _sca(aS, aD, cT, srcp, dstp, validp)
    numT = _scb(xpT, srcp.reshape(EP // C, C), dstp.reshape(EP // C, C), wT)

    out = _tc2(xpad, numT, bias.reshape(1, D), ln_gamma.reshape(1, D),
               ln_beta.reshape(1, D))
    return out[:, :N, :].reshape(b, t, n, d)


# speedup vs baseline: 33.0347x; 1.2174x over previous
import functools

import jax
import jax.numpy as jnp
from jax import lax
from jax.experimental import pallas as pl
from jax.experimental.pallas import tpu as pltpu
from jax.experimental.pallas import tpu_sc as plsc

F32 = jnp.float32
I32 = jnp.int32

G = 4
N = 10000
NP = 10240
D = 128
H = 2
RB = 1280
NRB = NP // RB

NSC = 2
NS = 16
C = 128
EL = 160000 + N
CPT = -(-EL // (NS * C))
EPT = CPT * C
EP = EPT * NS
RPT = NP // NS
RZ = RPT // 5


def _tc1_body(x_ref, w_ref, asrc_ref, adst_ref, xpT_ref, aT_ref, cT_ref):
    r = pl.program_id(1)
    xb = x_ref[0]
    xp = jnp.dot(xb, w_ref[...], preferred_element_type=F32)
    mx = []
    for h in range(H):
        xph = xp[:, h * D:(h + 1) * D]
        xpT_ref[h, 0] = xph
        a_s = (xph * asrc_ref[h][None, :]).sum(axis=1)
        a_d = (xph * adst_ref[h][None, :]).sum(axis=1)
        aT_ref[0, h, 0] = a_s.reshape(RB // D, D)
        aT_ref[1, h, 0] = a_d.reshape(RB // D, D)
        mx.append((jnp.max(a_s), jnp.max(a_d)))

    built = jnp.concatenate(
        [jnp.full((1, D), mx[0][0], F32), jnp.full((1, D), mx[1][0], F32),
         jnp.full((1, D), mx[0][1], F32), jnp.full((1, D), mx[1][1], F32),
         jnp.full((4, D), -1e30, F32)], axis=0)

    @pl.when(r == 0)
    def _():
        cT_ref[0] = jnp.full((8, D), -1e30, F32)
    cT_ref[0] = jnp.maximum(cT_ref[0], built)


_tc1 = pl.pallas_call(
    _tc1_body,
    grid=(G, NRB),
    in_specs=[
        pl.BlockSpec((1, RB, D), lambda g, r: (g, r, 0)),
        pl.BlockSpec((D, H * D), lambda g, r: (0, 0)),
        pl.BlockSpec((H, D), lambda g, r: (0, 0)),
        pl.BlockSpec((H, D), lambda g, r: (0, 0)),
    ],
    out_specs=[
        pl.BlockSpec((H, 1, RB, D), lambda g, r: (0, g, r, 0)),
        pl.BlockSpec((2, H, 1, RB // D, D), lambda g, r: (0, 0, g * NRB + r, 0, 0)),
        pl.BlockSpec((1, 8, D), lambda g, r: (g, 0, 0)),
    ],
    out_shape=[
        jax.ShapeDtypeStruct((H, G, NP, D), F32),
        jax.ShapeDtypeStruct((2, H, G * NRB, RB // D, D), F32),
        jax.ShapeDtypeStruct((G, 8, D), F32),
    ],
)


def _sca_body(aS, aD, cT, srcp, dstp, validp,
              wT, denP,
              src_v, dst_v, valid_v, wv_v, a_s_v, a_d_v, c_v, den_v):
    h = lax.axis_index("c")
    t = lax.axis_index("s")

    pltpu.sync_copy(srcp.at[pl.ds(t * EPT, EPT)], src_v)
    pltpu.sync_copy(dstp.at[pl.ds(t * EPT, EPT)], dst_v)
    pltpu.sync_copy(validp.at[pl.ds(t * EPT, EPT)], valid_v)

    for g in range(G):
        pltpu.sync_copy(aS.at[h, g], a_s_v)
        pltpu.sync_copy(aD.at[h, g], a_d_v)
        pltpu.sync_copy(cT.at[g], c_v)
        cs = c_v[h, pl.ds(0, 16)]
        cd = c_v[2 + h, pl.ds(0, 16)]
        csum = cs + cd
        c16 = jnp.where(csum > 0, csum, 0.2 * csum)

        def _zd(i, _):
            den_v[pl.ds(i * 16, 16)] = jnp.zeros((16,), F32)
            return 0
        lax.fori_loop(0, NP // 16, _zd, 0)

        def _grp(i, _):
            sl = pl.ds(i * 16, 16)
            av = plsc.load_gather(a_s_v, [src_v[sl]])
            bv = plsc.load_gather(a_d_v, [dst_v[sl]])
            al = av + bv
            al = jnp.where(al > 0, al, 0.2 * al)
            wv = jnp.exp(al - c16) * valid_v[sl]
            wv_v[sl] = wv
            plsc.addupdate_scatter(den_v, [dst_v[sl]], wv)
            return 0
        lax.fori_loop(0, EPT // 16, _grp, 0)

        pltpu.sync_copy(wv_v, wT.at[h, g].at[pl.ds(t * EPT, EPT)])
        pltpu.sync_copy(den_v, denP.at[h, g, t])


_sca = functools.partial(
    pl.kernel,
    out_type=(jax.ShapeDtypeStruct((H, G, EP), F32),
              jax.ShapeDtypeStruct((H, G, NS, NP), F32)),
    mesh=plsc.VectorSubcoreMesh(core_axis_name="c", subcore_axis_name="s",
                                num_cores=NSC, num_subcores=NS),
    compiler_params=pltpu.CompilerParams(needs_layout_passes=False),
    scratch_types=[
        pltpu.VMEM((EPT,), I32),
        pltpu.VMEM((EPT,), I32),
        pltpu.VMEM((EPT,), F32),
        pltpu.VMEM((EPT,), F32),
        pltpu.VMEM((NP,), F32),
        pltpu.VMEM((NP,), F32),
        pltpu.VMEM((8, D), F32),
        pltpu.VMEM((NP,), F32),
    ],
)(_sca_body)


def _scb_body(xpT, srcp2, dstp2, wT, denP,
              numT,
              sidx_v, didx_v, w_v, rows_v, tmp_v, inv_v,
              num_sh, sem0, sem1):
    h = lax.axis_index("c")
    t = lax.axis_index("s")
    zero16 = jnp.zeros((16,), F32)
    sems = (sem0, sem1)

    def _load_idx(row, p):
        pltpu.sync_copy(srcp2.at[pl.ds(row, 1)], sidx_v.at[p])
        pltpu.sync_copy(dstp2.at[pl.ds(row, 1)], didx_v.at[p])
        pltpu.sync_copy(wT.at[h, g_ref[0]].at[pl.ds(row * C, C)], w_v.at[p])

    for g in range(G):
        g_ref = (g,)

        def _z0(i, _):
            rows_v[0, i // 8, pl.ds((i % 8) * 16, 16)] = zero16
            return 0
        lax.fori_loop(0, C * 8, _z0, 0)
        for j in range(5):
            sl = pl.ds(t * RPT + j * RZ, RZ)
            pltpu.sync_copy(rows_v.at[0], num_sh.at[sl])
        plsc.subcore_barrier()

        def _scale_scatter(p, row):
            def _s(e, _):
                ws = plsc.load_gather(w_v.at[p], [jnp.zeros((16,), I32) + e])
                for u in range(D // 16):
                    sl = pl.ds(u * 16, 16)
                    rows_v[p, e, sl] = rows_v[p, e, sl] * ws
                return 0
            lax.fori_loop(0, C, _s, 0)
            pltpu.sync_copy(rows_v.at[p], num_sh.at[didx_v.at[p].at[0]], add=True)

        base = t * CPT
        _load_idx(base, 0)
        pltpu.async_copy(xpT.at[h, g].at[sidx_v.at[0].at[0]], rows_v.at[0], sem0)

        def _pair(i, _):
            k0 = base + 2 * i
            pltpu.make_async_copy(xpT.at[h, g].at[pl.ds(0, C)], rows_v.at[0],
                                  sem0).wait()
            _load_idx(k0 + 1, 1)
            pltpu.async_copy(xpT.at[h, g].at[sidx_v.at[1].at[0]], rows_v.at[1],
                             sem1)
            _scale_scatter(0, k0)
            nxt = base + lax.rem(2 * i + 2, CPT)
            _load_idx(nxt, 0)
            pltpu.async_copy(xpT.at[h, g].at[sidx_v.at[0].at[0]], rows_v.at[0],
                             sem0)
            pltpu.make_async_copy(xpT.at[h, g].at[pl.ds(0, C)], rows_v.at[1],
                                  sem1).wait()
            _scale_scatter(1, k0 + 1)
            return 0
        lax.fori_loop(0, CPT // 2, _pair, 0)
        pltpu.make_async_copy(xpT.at[h, g].at[pl.ds(0, C)], rows_v.at[0],
                              sem0).wait()

        plsc.subcore_barrier()

        def _zi(i, _):
            inv_v[pl.ds(i * 16, 16)] = zero16
            return 0
        lax.fori_loop(0, RPT // 16, _zi, 0)
        for j in range(NS):
            pltpu.sync_copy(denP.at[h, g, j].at[pl.ds(t * RPT, RPT)], tmp_v)

            def _ac(i, _):
                sl = pl.ds(i * 16, 16)
                inv_v[sl] = inv_v[sl] + tmp_v[sl]
                return 0
            lax.fori_loop(0, RPT // 16, _ac, 0)

        def _iv(i, _):
            sl = pl.ds(i * 16, 16)
            inv_v[sl] = 1.0 / (inv_v[sl] + 1e-16)
            return 0
        lax.fori_loop(0, RPT // 16, _iv, 0)

        for j in range(5):
            sl = pl.ds(t * RPT + j * RZ, RZ)
            pltpu.sync_copy(num_sh.at[sl], rows_v.at[0])

            def _n(e, _):
                ws = plsc.load_gather(inv_v, [jnp.zeros((16,), I32) + (j * RZ + e)])
                for u in range(D // 16):
                    sl2 = pl.ds(u * 16, 16)
                    rows_v[0, e, sl2] = rows_v[0, e, sl2] * ws
                return 0
            lax.fori_loop(0, RZ, _n, 0)
            pltpu.sync_copy(rows_v.at[0], numT.at[h, g].at[sl])


_scb = functools.partial(
    pl.kernel,
    out_type=jax.ShapeDtypeStruct((H, G, NP, D), F32),
    mesh=plsc.VectorSubcoreMesh(core_axis_name="c", subcore_axis_name="s",
                                num_cores=NSC, num_subcores=NS),
    compiler_params=pltpu.CompilerParams(needs_layout_passes=False),
    scratch_types=[
        pltpu.VMEM((2, 1, C), I32),
        pltpu.VMEM((2, 1, C), I32),
        pltpu.VMEM((2, C), F32),
        pltpu.VMEM((2, C, D), F32),
        pltpu.VMEM((RPT,), F32),
        pltpu.VMEM((RPT,), F32),
        pltpu.VMEM_SHARED((NP, D), F32),
        pltpu.SemaphoreType.DMA,
        pltpu.SemaphoreType.DMA,
    ],
)(_scb_body)


def _tc2_body(x_ref, num_ref, bias_ref, g_ref, b_ref, out_ref):
    o = 0.5 * (num_ref[0, 0] + num_ref[1, 0]) + bias_ref[0][None, :]
    y = x_ref[0] + o
    mu = jnp.mean(y, axis=1, keepdims=True)
    var = jnp.mean((y - mu) ** 2, axis=1, keepdims=True)
    out_ref[0] = (y - mu) * lax.rsqrt(var + 1e-5) * g_ref[0][None, :] + b_ref[0][None, :]


_tc2 = pl.pallas_call(
    _tc2_body,
    grid=(G, NRB),
    in_specs=[
        pl.BlockSpec((1, RB, D), lambda g, r: (g, r, 0)),
        pl.BlockSpec((H, 1, RB, D), lambda g, r: (0, g, r, 0)),
        pl.BlockSpec((1, D), lambda g, r: (0, 0)),
        pl.BlockSpec((1, D), lambda g, r: (0, 0)),
        pl.BlockSpec((1, D), lambda g, r: (0, 0)),
    ],
    out_specs=pl.BlockSpec((1, RB, D), lambda g, r: (g, r, 0)),
    out_shape=jax.ShapeDtypeStruct((G, NP, D), F32),
)


def kernel(x, edge_index, W, att_src, att_dst, bias, ln_gamma, ln_beta):
    b, t, n, d = x.shape
    x4 = x.reshape(G, N, D)
    xpad = jnp.pad(x4, ((0, 0), (0, NP - N), (0, 0)))

    xpT, aT, cT = _tc1(xpad, W, att_src, att_dst)
    aS = aT[0].reshape(H, G, NP)
    aD = aT[1].reshape(H, G, NP)

    ar = jnp.arange(N, dtype=I32)
    ei = edge_index.astype(I32)
    src = jnp.concatenate([ei[0], ar])
    dst = jnp.concatenate([ei[1], ar])
    srcp = jnp.pad(src, (0, EP - EL))
    dstp = jnp.pad(dst, (0, EP - EL))
    validp = jnp.pad(jnp.ones((EL,), F32), (0, EP - EL))

    wT, denP = _sca(aS, aD, cT, srcp, dstp, validp)
    numT = _scb(xpT, srcp.reshape(EP // C, C), dstp.reshape(EP // C, C), wT, denP)

    out = _tc2(xpad, numT, bias.reshape(1, D), ln_gamma.reshape(1, D),
               ln_beta.reshape(1, D))
    return out[:, :N, :].reshape(b, t, n, d)


# speedup vs baseline: 35.3113x; 1.0689x over previous
import functools

import jax
import jax.numpy as jnp
from jax import lax
from jax.experimental import pallas as pl
from jax.experimental.pallas import tpu as pltpu
from jax.experimental.pallas import tpu_sc as plsc

F32 = jnp.float32
I32 = jnp.int32

G = 4
N = 10000
NP = 10240
D = 128
H = 2
RB = 1280
NRB = NP // RB

NSC = 2
NS = 16
C = 128
EL = 160000 + N
CPT = -(-EL // (NS * C))
EPT = CPT * C
EP = EPT * NS
RPT = NP // NS
RZ = RPT // 5


def _tc1_body(x_ref, w_ref, asrc_ref, adst_ref, xpT_ref, aT_ref, cT_ref):
    r = pl.program_id(1)
    xb = x_ref[0]
    xp = jnp.dot(xb, w_ref[...], preferred_element_type=F32)
    mx = []
    for h in range(H):
        xph = xp[:, h * D:(h + 1) * D]
        xpT_ref[h, 0] = xph
        a_s = (xph * asrc_ref[h][None, :]).sum(axis=1)
        a_d = (xph * adst_ref[h][None, :]).sum(axis=1)
        aT_ref[0, h, 0] = a_s.reshape(RB // D, D)
        aT_ref[1, h, 0] = a_d.reshape(RB // D, D)
        mx.append((jnp.max(a_s), jnp.max(a_d)))

    built = jnp.concatenate(
        [jnp.full((1, D), mx[0][0], F32), jnp.full((1, D), mx[1][0], F32),
         jnp.full((1, D), mx[0][1], F32), jnp.full((1, D), mx[1][1], F32),
         jnp.full((4, D), -1e30, F32)], axis=0)

    @pl.when(r == 0)
    def _():
        cT_ref[0] = jnp.full((8, D), -1e30, F32)
    cT_ref[0] = jnp.maximum(cT_ref[0], built)


_tc1 = pl.pallas_call(
    _tc1_body,
    grid=(G, NRB),
    in_specs=[
        pl.BlockSpec((1, RB, D), lambda g, r: (g, r, 0)),
        pl.BlockSpec((D, H * D), lambda g, r: (0, 0)),
        pl.BlockSpec((H, D), lambda g, r: (0, 0)),
        pl.BlockSpec((H, D), lambda g, r: (0, 0)),
    ],
    out_specs=[
        pl.BlockSpec((H, 1, RB, D), lambda g, r: (0, g, r, 0)),
        pl.BlockSpec((2, H, 1, RB // D, D), lambda g, r: (0, 0, g * NRB + r, 0, 0)),
        pl.BlockSpec((1, 8, D), lambda g, r: (g, 0, 0)),
    ],
    out_shape=[
        jax.ShapeDtypeStruct((H, G, NP, D), F32),
        jax.ShapeDtypeStruct((2, H, G * NRB, RB // D, D), F32),
        jax.ShapeDtypeStruct((G, 8, D), F32),
    ],
)


def _sca_body(aS, aD, cT, srcp, dstp, validp,
              wT, denP,
              src_v, dst_v, valid_v, wv_v, a_s_v, a_d_v, c_v, den_v):
    h = lax.axis_index("c")
    t = lax.axis_index("s")

    pltpu.sync_copy(srcp.at[pl.ds(t * EPT, EPT)], src_v)
    pltpu.sync_copy(dstp.at[pl.ds(t * EPT, EPT)], dst_v)
    pltpu.sync_copy(validp.at[pl.ds(t * EPT, EPT)], valid_v)

    for g in range(G):
        pltpu.sync_copy(aS.at[h, g], a_s_v)
        pltpu.sync_copy(aD.at[h, g], a_d_v)
        pltpu.sync_copy(cT.at[g], c_v)
        cs = c_v[h, pl.ds(0, 16)]
        cd = c_v[2 + h, pl.ds(0, 16)]
        csum = cs + cd
        c16 = jnp.where(csum > 0, csum, 0.2 * csum)

        def _zd(i, _):
            den_v[pl.ds(i * 16, 16)] = jnp.zeros((16,), F32)
            return 0
        lax.fori_loop(0, NP // 16, _zd, 0)

        def _grp(i, _):
            sl = pl.ds(i * 16, 16)
            av = plsc.load_gather(a_s_v, [src_v[sl]])
            bv = plsc.load_gather(a_d_v, [dst_v[sl]])
            al = av + bv
            al = jnp.where(al > 0, al, 0.2 * al)
            wv = jnp.exp(al - c16) * valid_v[sl]
            wv_v[sl] = wv
            plsc.addupdate_scatter(den_v, [dst_v[sl]], wv)
            return 0
        lax.fori_loop(0, EPT // 16, _grp, 0)

        pltpu.sync_copy(wv_v, wT.at[h, g].at[pl.ds(t * EPT, EPT)])
        pltpu.sync_copy(den_v, denP.at[h, g, t])


_sca = functools.partial(
    pl.kernel,
    out_type=(jax.ShapeDtypeStruct((H, G, EP), F32),
              jax.ShapeDtypeStruct((H, G, NS, NP), F32)),
    mesh=plsc.VectorSubcoreMesh(core_axis_name="c", subcore_axis_name="s",
                                num_cores=NSC, num_subcores=NS),
    compiler_params=pltpu.CompilerParams(needs_layout_passes=False),
    scratch_types=[
        pltpu.VMEM((EPT,), I32),
        pltpu.VMEM((EPT,), I32),
        pltpu.VMEM((EPT,), F32),
        pltpu.VMEM((EPT,), F32),
        pltpu.VMEM((NP,), F32),
        pltpu.VMEM((NP,), F32),
        pltpu.VMEM((8, D), F32),
        pltpu.VMEM((NP,), F32),
    ],
)(_sca_body)


def _scb_body(xpT, srcp2, dstp2, wT, denP,
              numT,
              sidx_v, didx_v, w_v, rows_v, tmp_v, inv_v,
              num_sh, sem0, sem1):
    h = lax.axis_index("c")
    t = lax.axis_index("s")
    zero16 = jnp.zeros((16,), F32)
    sems = (sem0, sem1)

    def _load_idx(row, p):
        pltpu.sync_copy(srcp2.at[pl.ds(row, 1)], sidx_v.at[p])
        pltpu.sync_copy(dstp2.at[pl.ds(row, 1)], didx_v.at[p])
        pltpu.sync_copy(wT.at[h, g_ref[0]].at[pl.ds(row * C, C)], w_v.at[p])

    for g in range(G):
        g_ref = (g,)

        def _z0(i, _):
            rows_v[0, i // 8, pl.ds((i % 8) * 16, 16)] = zero16
            return 0
        lax.fori_loop(0, C * 8, _z0, 0)
        for j in range(5):
            sl = pl.ds(t * RPT + j * RZ, RZ)
            pltpu.sync_copy(rows_v.at[0], num_sh.at[sl])
        plsc.subcore_barrier()

        def _scale_scatter(p, row):
            def _s(i, _):
                e0 = i * 2
                ws0 = plsc.load_gather(w_v.at[p], [jnp.zeros((16,), I32) + e0])
                ws1 = plsc.load_gather(w_v.at[p], [jnp.zeros((16,), I32) + (e0 + 1)])
                for u in range(D // 16):
                    sl = pl.ds(u * 16, 16)
                    rows_v[p, e0, sl] = rows_v[p, e0, sl] * ws0
                    rows_v[p, e0 + 1, sl] = rows_v[p, e0 + 1, sl] * ws1
                return 0
            lax.fori_loop(0, C // 2, _s, 0)
            pltpu.sync_copy(rows_v.at[p], num_sh.at[didx_v.at[p].at[0]], add=True)

        base = t * CPT
        _load_idx(base, 0)
        pltpu.async_copy(xpT.at[h, g].at[sidx_v.at[0].at[0]], rows_v.at[0], sem0)

        def _pair(i, _):
            k0 = base + 2 * i
            pltpu.make_async_copy(xpT.at[h, g].at[pl.ds(0, C)], rows_v.at[0],
                                  sem0).wait()
            _load_idx(k0 + 1, 1)
            pltpu.async_copy(xpT.at[h, g].at[sidx_v.at[1].at[0]], rows_v.at[1],
                             sem1)
            _scale_scatter(0, k0)
            nxt = base + lax.rem(2 * i + 2, CPT)
            _load_idx(nxt, 0)
            pltpu.async_copy(xpT.at[h, g].at[sidx_v.at[0].at[0]], rows_v.at[0],
                             sem0)
            pltpu.make_async_copy(xpT.at[h, g].at[pl.ds(0, C)], rows_v.at[1],
                                  sem1).wait()
            _scale_scatter(1, k0 + 1)
            return 0
        lax.fori_loop(0, CPT // 2, _pair, 0)
        pltpu.make_async_copy(xpT.at[h, g].at[pl.ds(0, C)], rows_v.at[0],
                              sem0).wait()

        plsc.subcore_barrier()

        def _zi(i, _):
            inv_v[pl.ds(i * 16, 16)] = zero16
            return 0
        lax.fori_loop(0, RPT // 16, _zi, 0)
        for j in range(NS):
            pltpu.sync_copy(denP.at[h, g, j].at[pl.ds(t * RPT, RPT)], tmp_v)

            def _ac(i, _):
                sl = pl.ds(i * 16, 16)
                inv_v[sl] = inv_v[sl] + tmp_v[sl]
                return 0
            lax.fori_loop(0, RPT // 16, _ac, 0)

        def _iv(i, _):
            sl = pl.ds(i * 16, 16)
            inv_v[sl] = 1.0 / (inv_v[sl] + 1e-16)
            return 0
        lax.fori_loop(0, RPT // 16, _iv, 0)

        for j in range(5):
            sl = pl.ds(t * RPT + j * RZ, RZ)
            pltpu.sync_copy(num_sh.at[sl], rows_v.at[0])

            def _n(i, _):
                e0 = i * 2
                ws0 = plsc.load_gather(inv_v, [jnp.zeros((16,), I32) + (j * RZ + e0)])
                ws1 = plsc.load_gather(inv_v, [jnp.zeros((16,), I32) + (j * RZ + e0 + 1)])
                for u in range(D // 16):
                    sl2 = pl.ds(u * 16, 16)
                    rows_v[0, e0, sl2] = rows_v[0, e0, sl2] * ws0
                    rows_v[0, e0 + 1, sl2] = rows_v[0, e0 + 1, sl2] * ws1
                return 0
            lax.fori_loop(0, RZ // 2, _n, 0)
            pltpu.sync_copy(rows_v.at[0], numT.at[h, g].at[sl])


_scb = functools.partial(
    pl.kernel,
    out_type=jax.ShapeDtypeStruct((H, G, NP, D), F32),
    mesh=plsc.VectorSubcoreMesh(core_axis_name="c", subcore_axis_name="s",
                                num_cores=NSC, num_subcores=NS),
    compiler_params=pltpu.CompilerParams(needs_layout_passes=False),
    scratch_types=[
        pltpu.VMEM((2, 1, C), I32),
        pltpu.VMEM((2, 1, C), I32),
        pltpu.VMEM((2, C), F32),
        pltpu.VMEM((2, C, D), F32),
        pltpu.VMEM((RPT,), F32),
        pltpu.VMEM((RPT,), F32),
        pltpu.VMEM_SHARED((NP, D), F32),
        pltpu.SemaphoreType.DMA,
        pltpu.SemaphoreType.DMA,
    ],
)(_scb_body)


def _tc2_body(x_ref, num_ref, bias_ref, g_ref, b_ref, out_ref):
    o = 0.5 * (num_ref[0, 0] + num_ref[1, 0]) + bias_ref[0][None, :]
    y = x_ref[0] + o
    mu = jnp.mean(y, axis=1, keepdims=True)
    var = jnp.mean((y - mu) ** 2, axis=1, keepdims=True)
    out_ref[0] = (y - mu) * lax.rsqrt(var + 1e-5) * g_ref[0][None, :] + b_ref[0][None, :]


_tc2 = pl.pallas_call(
    _tc2_body,
    grid=(G, NRB),
    in_specs=[
        pl.BlockSpec((1, RB, D), lambda g, r: (g, r, 0)),
        pl.BlockSpec((H, 1, RB, D), lambda g, r: (0, g, r, 0)),
        pl.BlockSpec((1, D), lambda g, r: (0, 0)),
        pl.BlockSpec((1, D), lambda g, r: (0, 0)),
        pl.BlockSpec((1, D), lambda g, r: (0, 0)),
    ],
    out_specs=pl.BlockSpec((1, RB, D), lambda g, r: (g, r, 0)),
    out_shape=jax.ShapeDtypeStruct((G, NP, D), F32),
)


def kernel(x, edge_index, W, att_src, att_dst, bias, ln_gamma, ln_beta):
    b, t, n, d = x.shape
    x4 = x.reshape(G, N, D)
    xpad = jnp.pad(x4, ((0, 0), (0, NP - N), (0, 0)))

    xpT, aT, cT = _tc1(xpad, W, att_src, att_dst)
    aS = aT[0].reshape(H, G, NP)
    aD = aT[1].reshape(H, G, NP)

    ar = jnp.arange(N, dtype=I32)
    ei = edge_index.astype(I32)
    src = jnp.concatenate([ei[0], ar])
    dst = jnp.concatenate([ei[1], ar])
    srcp = jnp.pad(src, (0, EP - EL))
    dstp = jnp.pad(dst, (0, EP - EL))
    validp = jnp.pad(jnp.ones((EL,), F32), (0, EP - EL))

    wT, denP = _sca(aS, aD, cT, srcp, dstp, validp)
    numT = _scb(xpT, srcp.reshape(EP // C, C), dstp.reshape(EP // C, C), wT, denP)

    out = _tc2(xpad, numT, bias.reshape(1, D), ln_gamma.reshape(1, D),
               ln_beta.reshape(1, D))
    return out[:, :N, :].reshape(b, t, n, d)
